# Initial kernel scaffold; baseline (speedup 1.0000x reference)
import jax
import jax.numpy as jnp
from jax.experimental import pallas as pl


def kernel(e_two, batch_route_ids, W1, b1, W2, b2):
    raise NotImplementedError("write your pallas kernel here")


# speedup vs baseline: 1.4281x; 1.4281x over previous
import functools

import jax
import jax.numpy as jnp
from jax.experimental import pallas as pl
from jax.experimental.pallas import tpu as pltpu


def _router_body(meta_ref, starts_ref, ends_ref,
                 x_ref, w1_ref, b1_ref, w2_ref, b2_ref, out_ref, *, tile):
    w = pl.program_id(0)
    t = meta_ref[0, w]
    e = meta_ref[1, w]
    valid = meta_ref[2, w]
    first = meta_ref[3, w]

    @pl.when(first == 1)
    def _():
        out_ref[...] = jnp.zeros_like(out_ref)

    @pl.when(valid == 1)
    def _():
        xv = x_ref[...]
        h = jnp.dot(xv, w1_ref[0], preferred_element_type=jnp.float32)
        h = jnp.maximum(h + b1_ref[0], 0.0)
        logits = jnp.dot(h, w2_ref[0], preferred_element_type=jnp.float32)
        logits = logits + b2_ref[0]
        m = jnp.max(logits, axis=-1, keepdims=True)
        p = jnp.exp(logits - m)
        p = p / jnp.sum(p, axis=-1, keepdims=True)
        gidx = t * tile + jax.lax.broadcasted_iota(jnp.int32, (tile, 1), 0)
        mask = (gidx >= starts_ref[e]) & (gidx < ends_ref[e])
        out_ref[...] += jnp.where(mask, p, 0.0)


def kernel(e_two, batch_route_ids, W1, b1, W2, b2):
    Bq, Sq, D = e_two.shape
    E, _, F = W1.shape
    V = W2.shape[2]
    T = Bq * Sq
    TILE = min(512, T)
    nt = T // TILE
    NWI = nt + E - 1

    x = e_two.reshape(T, D)
    rid = batch_route_ids.reshape(T)

    perm = jnp.argsort(rid)
    inv = jnp.zeros((T,), jnp.int32).at[perm].set(
        jnp.arange(T, dtype=jnp.int32))
    x_sorted = jnp.take(x, perm, axis=0)

    g = jnp.bincount(rid, length=E).astype(jnp.int32)
    ends = jnp.cumsum(g)
    starts = ends - g

    first_tile = starts // TILE
    last_tile = jnp.where(g > 0, (ends - 1) // TILE, 0)
    ntile_e = jnp.where(g > 0, last_tile - first_tile + 1, 0)
    item_end = jnp.cumsum(ntile_e)
    item_start = item_end - ntile_e
    total = item_end[E - 1]

    wids = jnp.arange(NWI, dtype=jnp.int32)
    e_of_w = jnp.sum(wids[:, None] >= item_end[None, :], axis=1)
    e_of_w = jnp.minimum(e_of_w, E - 1).astype(jnp.int32)
    tile_w = jnp.take(first_tile, e_of_w) + (wids - jnp.take(item_start, e_of_w))
    validm = wids < total
    last_idx = jnp.maximum(total - 1, 0)
    tile_w = jnp.where(validm, tile_w, jnp.take(tile_w, last_idx))
    route_w = jnp.where(validm, e_of_w, jnp.take(e_of_w, last_idx))
    prev_tile = jnp.concatenate(
        [jnp.full((1,), -1, jnp.int32), tile_w[:-1]])
    is_first = (tile_w != prev_tile).astype(jnp.int32)
    meta = jnp.stack([tile_w, route_w, validm.astype(jnp.int32), is_first])

    grid_spec = pltpu.PrefetchScalarGridSpec(
        num_scalar_prefetch=3,
        grid=(NWI,),
        in_specs=[
            pl.BlockSpec((TILE, D), lambda w, meta, s, e: (meta[0, w], 0)),
            pl.BlockSpec((1, D, F), lambda w, meta, s, e: (meta[1, w], 0, 0)),
            pl.BlockSpec((1, 1, F), lambda w, meta, s, e: (meta[1, w], 0, 0)),
            pl.BlockSpec((1, F, V), lambda w, meta, s, e: (meta[1, w], 0, 0)),
            pl.BlockSpec((1, 1, V), lambda w, meta, s, e: (meta[1, w], 0, 0)),
        ],
        out_specs=pl.BlockSpec((TILE, V), lambda w, meta, s, e: (meta[0, w], 0)),
    )
    probs_sorted = pl.pallas_call(
        functools.partial(_router_body, tile=TILE),
        grid_spec=grid_spec,
        out_shape=jax.ShapeDtypeStruct((T, V), jnp.float32),
        compiler_params=pltpu.CompilerParams(
            dimension_semantics=("arbitrary",)),
    )(meta, starts, ends, x_sorted, W1,
      b1.reshape(E, 1, F), W2, b2.reshape(E, 1, V))

    out = jnp.take(probs_sorted, inv, axis=0)
    return out.reshape(Bq, Sq, V)


# speedup vs baseline: 1.6445x; 1.1516x over previous
import functools

import jax
import jax.numpy as jnp
from jax.experimental import pallas as pl
from jax.experimental.pallas import tpu as pltpu


def _router_body(meta_ref, starts_ref, ends_ref,
                 x_ref, w1_ref, b1_ref, w2_ref, b2_ref, out_ref, *, tile):
    w = pl.program_id(0)
    t = meta_ref[0, w]
    e = meta_ref[1, w]
    valid = meta_ref[2, w]
    first = meta_ref[3, w]

    @pl.when(first == 1)
    def _():
        out_ref[...] = jnp.zeros_like(out_ref)

    @pl.when(valid == 1)
    def _():
        xv = x_ref[...]
        h = jnp.dot(xv, w1_ref[0], preferred_element_type=jnp.float32)
        h = jnp.maximum(h + b1_ref[0], 0.0)
        logits = jnp.dot(h, w2_ref[0], preferred_element_type=jnp.float32)
        logits = logits + b2_ref[0]
        m = jnp.max(logits, axis=-1, keepdims=True)
        p = jnp.exp(logits - m)
        p = p / jnp.sum(p, axis=-1, keepdims=True)
        gidx = t * tile + jax.lax.broadcasted_iota(jnp.int32, (tile, 1), 0)
        mask = (gidx >= starts_ref[e]) & (gidx < ends_ref[e])
        out_ref[...] += jnp.where(mask, p, 0.0)


def kernel(e_two, batch_route_ids, W1, b1, W2, b2):
    Bq, Sq, D = e_two.shape
    E, _, F = W1.shape
    V = W2.shape[2]
    T = Bq * Sq
    TILE = min(512, T)
    nt = T // TILE
    NWI = nt + E - 1

    x = e_two.reshape(T, D)
    rid = batch_route_ids.reshape(T)

    CH = 128
    NC = T // CH
    oh = (rid[None, :] == jnp.arange(E, dtype=jnp.int32)[:, None])
    ohf = oh.astype(jnp.float32).reshape(E, NC, CH)
    tril_ch = (jnp.arange(CH)[:, None] < jnp.arange(CH)[None, :]).astype(
        jnp.float32)
    within = jnp.einsum('ecm,ml->ecl', ohf, tril_ch)
    tot = jnp.sum(ohf, axis=2)
    tril_nc = (jnp.arange(NC)[:, None] < jnp.arange(NC)[None, :]).astype(
        jnp.float32)
    offs = jnp.einsum('em,ml->el', tot, tril_nc)
    rank = (within + offs[:, :, None]).reshape(E, T)
    g = jnp.sum(tot, axis=1).astype(jnp.int32)
    ends = jnp.cumsum(g)
    starts = ends - g
    pos_f = jnp.sum(jnp.where(oh, rank + starts[:, None].astype(jnp.float32),
                              0.0), axis=0)
    inv = pos_f.astype(jnp.int32)

    x_sorted = jnp.zeros((T, D), jnp.float32).at[inv].set(
        x, unique_indices=True, mode='promise_in_bounds')

    first_tile = starts // TILE
    last_tile = jnp.where(g > 0, (ends - 1) // TILE, 0)
    ntile_e = jnp.where(g > 0, last_tile - first_tile + 1, 0)
    item_end = jnp.cumsum(ntile_e)
    item_start = item_end - ntile_e
    total = item_end[E - 1]

    wids = jnp.arange(NWI, dtype=jnp.int32)
    e_of_w = jnp.sum(wids[:, None] >= item_end[None, :], axis=1)
    e_of_w = jnp.minimum(e_of_w, E - 1).astype(jnp.int32)
    tile_w = jnp.take(first_tile, e_of_w) + (wids - jnp.take(item_start, e_of_w))
    validm = wids < total
    last_idx = jnp.maximum(total - 1, 0)
    tile_w = jnp.where(validm, tile_w, jnp.take(tile_w, last_idx))
    route_w = jnp.where(validm, e_of_w, jnp.take(e_of_w, last_idx))
    prev_tile = jnp.concatenate(
        [jnp.full((1,), -1, jnp.int32), tile_w[:-1]])
    is_first = (tile_w != prev_tile).astype(jnp.int32)
    meta = jnp.stack([tile_w, route_w, validm.astype(jnp.int32), is_first])

    grid_spec = pltpu.PrefetchScalarGridSpec(
        num_scalar_prefetch=3,
        grid=(NWI,),
        in_specs=[
            pl.BlockSpec((TILE, D), lambda w, meta, s, e: (meta[0, w], 0)),
            pl.BlockSpec((1, D, F), lambda w, meta, s, e: (meta[1, w], 0, 0)),
            pl.BlockSpec((1, 1, F), lambda w, meta, s, e: (meta[1, w], 0, 0)),
            pl.BlockSpec((1, F, V), lambda w, meta, s, e: (meta[1, w], 0, 0)),
            pl.BlockSpec((1, 1, V), lambda w, meta, s, e: (meta[1, w], 0, 0)),
        ],
        out_specs=pl.BlockSpec((TILE, V), lambda w, meta, s, e: (meta[0, w], 0)),
    )
    probs_sorted = pl.pallas_call(
        functools.partial(_router_body, tile=TILE),
        grid_spec=grid_spec,
        out_shape=jax.ShapeDtypeStruct((T, V), jnp.float32),
        compiler_params=pltpu.CompilerParams(
            dimension_semantics=("arbitrary",)),
    )(meta, starts, ends, x_sorted, W1,
      b1.reshape(E, 1, F), W2, b2.reshape(E, 1, V))

    out = probs_sorted.at[inv].get(unique_indices=True,
                                   mode='promise_in_bounds')
    return out.reshape(Bq, Sq, V)
